# Initial kernel scaffold; baseline (speedup 1.0000x reference)
import jax
import jax.numpy as jnp
from jax.experimental import pallas as pl


def kernel(x_seq, edge_index, node_embeddings, W1, b1, W_ih, W_hh, b_ih, b_hh, W2, b2):
    raise NotImplementedError("write your pallas kernel here")


# speedup vs baseline: 34.0657x; 34.0657x over previous
import functools

import jax
import jax.numpy as jnp
from jax import lax
from jax.experimental import pallas as pl
from jax.experimental.pallas import tpu as pltpu
from jax.experimental.pallas import tpu_sc as plsc

N = 50000
T = 12
TP = 16
HID = 32
NC, NS, L = 2, 16, 16
NW = NC * NS
NPAD = 50176
NSL = NPAD // NS
NWSL = NPAD // NW
NE = 800000
NEPAD = 802816
EPW = NEPAD // NW
NE2 = NE + N
NE2PAD = 851968
EPW2 = NE2PAD // NW
CH = 128

_mesh = plsc.VectorSubcoreMesh(core_axis_name="c", subcore_axis_name="s",
                               num_cores=NC, num_subcores=NS)
_sc_params = pltpu.CompilerParams(needs_layout_passes=False, use_tc_tiling_on_sc=False)
_f32 = jnp.float32
_i32 = jnp.int32


def _wid():
    return lax.axis_index("c") * NS + lax.axis_index("s")


def _vec_loop(n, body):
    lax.fori_loop(0, n, lambda k, c: (body(k), c)[1], 0)


def _sl16(k):
    return pl.ds(pl.multiple_of(k * L, L), L)


def _rows_to_flat(r2, fl):
    for j in range(CH):
        fl[pl.ds(j * TP, TP)] = r2[j, :]


def _flat_to_rows(fl, r2):
    for j in range(CH):
        r2[j, :] = fl[pl.ds(j * TP, TP)]


EMBED = 10


def _edge_p_body(srcp, dstp, emb16, zn, p_out, s_p0, s_p1, s_sh,
                 zb, srcv, dstv, rows_s, rows_t, fs, ft, p_b):
    c = lax.axis_index("c")
    s = lax.axis_index("s")
    wid = _wid()
    nsl = pl.ds(pl.multiple_of(s * NSL, 8), NSL)
    pltpu.sync_copy(zn.at[nsl], zb)
    pltpu.sync_copy(zb, s_sh.at[nsl])
    plsc.subcore_barrier()

    def chunk(i, carry):
        off = pl.multiple_of(wid * EPW + i * CH, 8)
        esl = pl.ds(off, CH)
        pltpu.sync_copy(srcp.at[esl], srcv)
        pltpu.sync_copy(dstp.at[esl], dstv)
        pltpu.sync_copy(emb16.at[srcv], rows_s)
        pltpu.sync_copy(emb16.at[dstv], rows_t)
        _rows_to_flat(rows_s, fs)
        _rows_to_flat(rows_t, ft)

        iota = lax.iota(_i32, L)
        for k in range(CH // L):
            sl = _sl16(k)
            fidx0 = iota * TP + (k * L * TP)
            dv = jnp.zeros((L,), _f32)
            for e in range(EMBED):
                dv = dv + (plsc.load_gather(fs, [fidx0 + e])
                           * plsc.load_gather(ft, [fidx0 + e]))
            pv = jnp.exp(jnp.maximum(dv, 0.0))
            ids = off + (k * L) + iota
            p_b[sl] = jnp.where(ids < NE, pv, 0.0)
        pltpu.sync_copy(p_b, p_out.at[esl])
        pltpu.sync_copy(p_b, s_sh.at[srcv], add=True)
        return carry
    lax.fori_loop(0, EPW // CH, chunk, 0)
    plsc.subcore_barrier()

    @pl.when(c == 0)
    def _():
        pltpu.sync_copy(s_sh.at[nsl], s_p0.at[nsl])

    @pl.when(c == 1)
    def _():
        pltpu.sync_copy(s_sh.at[nsl], s_p1.at[nsl])


def _edge_w_body(srcp, dstp, p_in, s_p0, s_p1, zn, w_out, deg_p0, deg_p1,
                 deg_sh, tab0, tab1, srcv, dstv, p_b, w_b):
    c = lax.axis_index("c")
    s = lax.axis_index("s")
    wid = _wid()
    nsl = pl.ds(pl.multiple_of(s * NSL, 8), NSL)
    zsl = pl.ds(0, NSL)
    pltpu.sync_copy(zn.at[nsl], tab1.at[zsl])
    pltpu.sync_copy(tab1.at[zsl], deg_sh.at[nsl])
    pltpu.sync_copy(s_p0, tab0)
    pltpu.sync_copy(s_p1, tab1)

    def comb(k):
        sl = _sl16(k)
        tab0[sl] = tab0[sl] + tab1[sl] + 1e-8
    _vec_loop(NPAD // L, comb)
    plsc.subcore_barrier()

    def chunk(i, carry):
        off = pl.multiple_of(wid * EPW + i * CH, 8)
        esl = pl.ds(off, CH)
        pltpu.sync_copy(srcp.at[esl], srcv)
        pltpu.sync_copy(dstp.at[esl], dstv)
        pltpu.sync_copy(p_in.at[esl], p_b)
        for k in range(CH // L):
            sl = _sl16(k)
            sv = plsc.load_gather(tab0, [srcv[sl]])
            w_b[sl] = p_b[sl] / sv
        pltpu.sync_copy(w_b, w_out.at[esl])
        pltpu.sync_copy(w_b, deg_sh.at[dstv], add=True)
        return carry
    lax.fori_loop(0, EPW // CH, chunk, 0)
    plsc.subcore_barrier()

    @pl.when(c == 0)
    def _():
        pltpu.sync_copy(deg_sh.at[nsl], deg_p0.at[nsl])

    @pl.when(c == 1)
    def _():
        pltpu.sync_copy(deg_sh.at[nsl], deg_p1.at[nsl])


def _dis_body(deg_p0, deg_p1, dis_out, d0, d1):
    wid = _wid()
    nsl = pl.ds(pl.multiple_of(wid * NWSL, 8), NWSL)
    pltpu.sync_copy(deg_p0.at[nsl], d0)
    pltpu.sync_copy(deg_p1.at[nsl], d1)

    def vec(k):
        sl = _sl16(k)
        deg = d0[sl] + d1[sl] + 1.0
        xi = lax.bitcast_convert_type(deg, _i32)
        y = lax.bitcast_convert_type(0x5F3759DF - lax.shift_right_logical(xi, 1), _f32)
        h = deg * 0.5
        y = y * (1.5 - h * y * y)
        y = y * (1.5 - h * y * y)
        y = y * (1.5 - h * y * y)
        y = y * (1.5 - h * y * y)
        d0[sl] = y
    _vec_loop(NWSL // L, vec)
    pltpu.sync_copy(d0, dis_out.at[nsl])


def _spmv_body(srcx, dstx, wx, dis_in, x16, znt, agg_p0, agg_p1, norm_out,
               agg_sh, dis_tab, zb2, srcv, dstv, w_b, norm_b, rows, fr):
    c = lax.axis_index("c")
    s = lax.axis_index("s")
    wid = _wid()
    pltpu.sync_copy(dis_in, dis_tab)
    nsl = pl.ds(pl.multiple_of(s * NSL, 8), NSL)
    for q in range(8):
        rsl = pl.ds(pl.multiple_of(s * NSL + q * (NSL // 8), 8), NSL // 8)
        pltpu.sync_copy(znt.at[rsl], zb2)
        pltpu.sync_copy(zb2, agg_sh.at[rsl])
    plsc.subcore_barrier()

    def chunk(i, carry):
        off = pl.multiple_of(wid * EPW2 + i * CH, 8)
        esl = pl.ds(off, CH)
        pltpu.sync_copy(srcx.at[esl], srcv)
        pltpu.sync_copy(dstx.at[esl], dstv)
        pltpu.sync_copy(wx.at[esl], w_b)
        pltpu.sync_copy(x16.at[srcv], rows)
        _rows_to_flat(rows, fr)
        iota = lax.iota(_i32, L)
        for k in range(CH // L):
            sl = _sl16(k)
            dsv = plsc.load_gather(dis_tab, [srcv[sl]])
            ddv = plsc.load_gather(dis_tab, [dstv[sl]])
            nv = dsv * w_b[sl] * ddv
            norm_b[sl] = nv
            fidx0 = iota * TP + (k * L * TP)
            for t in range(T):
                col = plsc.load_gather(fr, [fidx0 + t])
                plsc.store_scatter(fr, [fidx0 + t], col * nv)
        _flat_to_rows(fr, rows)
        pltpu.sync_copy(norm_b, norm_out.at[esl])
        pltpu.sync_copy(rows, agg_sh.at[dstv], add=True)
        return carry
    lax.fori_loop(0, EPW2 // CH, chunk, 0)
    plsc.subcore_barrier()

    @pl.when(c == 0)
    def _():
        pltpu.sync_copy(agg_sh.at[nsl], agg_p0.at[nsl])

    @pl.when(c == 1)
    def _():
        pltpu.sync_copy(agg_sh.at[nsl], agg_p1.at[nsl])


def _final_body(srcx, dstx, norm_in, hw_in, zn, out_p0, out_p1, out_sh,
                hw_tab, srcv, dstv, norm_b, val_b):
    c = lax.axis_index("c")
    s = lax.axis_index("s")
    wid = _wid()
    nsl = pl.ds(pl.multiple_of(s * NSL, 8), NSL)
    zsl = pl.ds(0, NSL)
    pltpu.sync_copy(zn.at[nsl], hw_tab.at[zsl])
    pltpu.sync_copy(hw_tab.at[zsl], out_sh.at[nsl])
    pltpu.sync_copy(hw_in, hw_tab)
    plsc.subcore_barrier()

    def chunk(i, carry):
        off = pl.multiple_of(wid * EPW2 + i * CH, 8)
        esl = pl.ds(off, CH)
        pltpu.sync_copy(srcx.at[esl], srcv)
        pltpu.sync_copy(dstx.at[esl], dstv)
        pltpu.sync_copy(norm_in.at[esl], norm_b)
        for k in range(CH // L):
            sl = _sl16(k)
            hv = plsc.load_gather(hw_tab, [srcv[sl]])
            val_b[sl] = norm_b[sl] * hv
        pltpu.sync_copy(val_b, out_sh.at[dstv], add=True)
        return carry
    lax.fori_loop(0, EPW2 // CH, chunk, 0)
    plsc.subcore_barrier()

    @pl.when(c == 0)
    def _():
        pltpu.sync_copy(out_sh.at[nsl], out_p0.at[nsl])

    @pl.when(c == 1)
    def _():
        pltpu.sync_copy(out_sh.at[nsl], out_p1.at[nsl])


def _gru_body(agg0, agg1, w1, b1, wiht, whht, bih, bhh, w2, hw_out):
    agg = agg0[:] + agg1[:]
    bn = agg.shape[0]
    h = jnp.zeros((bn, HID), _f32)
    for t in range(T):
        x = jnp.maximum(agg[:, t:t + 1] * w1[:] + b1[:], 0.0)
        gi = jnp.dot(x, wiht[:], preferred_element_type=_f32) + bih[:]
        gh = jnp.dot(h, whht[:], preferred_element_type=_f32) + bhh[:]
        r = jax.nn.sigmoid(gi[:, :HID] + gh[:, :HID])
        z = jax.nn.sigmoid(gi[:, HID:2 * HID] + gh[:, HID:2 * HID])
        n = jnp.tanh(gi[:, 2 * HID:] + r * gh[:, 2 * HID:])
        h = (1.0 - z) * n + z * h
    hw_out[:, :] = jnp.dot(h, w2[:], preferred_element_type=_f32)


def _comb_body(f0, f1, b2, out):
    out[:, :] = f0[:] + f1[:] + b2[0, 0]


def kernel(x_seq, edge_index, node_embeddings, W1, b1, W_ih, W_hh, b_ih,
           b_hh, W2, b2):
    f32 = _f32
    src = edge_index[0].astype(_i32)
    dst = edge_index[1].astype(_i32)
    srcp = jnp.pad(src, (0, NEPAD - NE))
    dstp = jnp.pad(dst, (0, NEPAD - NE))
    emb16 = jnp.pad(node_embeddings.astype(f32), ((0, 0), (0, TP - node_embeddings.shape[1])))
    x16 = jnp.pad(x_seq.astype(f32), ((0, 0), (0, TP - T)))
    zn = jnp.zeros((NPAD,), f32)
    znt = jnp.zeros((NPAD, TP), f32)

    sds = jax.ShapeDtypeStruct
    p, s_p0, s_p1 = pl.kernel(
        _edge_p_body,
        compiler_params=_sc_params,
        out_type=[sds((NEPAD,), f32), sds((NPAD,), f32), sds((NPAD,), f32)],
        mesh=_mesh,
        scratch_types=[
            pltpu.VMEM_SHARED((NPAD,), f32),
            pltpu.VMEM((NSL,), f32),
            pltpu.VMEM((CH,), _i32), pltpu.VMEM((CH,), _i32),
            pltpu.VMEM((CH, TP), f32), pltpu.VMEM((CH, TP), f32),
            pltpu.VMEM((CH * TP,), f32), pltpu.VMEM((CH * TP,), f32),
            pltpu.VMEM((CH,), f32),
        ],
    )(srcp, dstp, emb16, zn)

    w, deg_p0, deg_p1 = pl.kernel(
        _edge_w_body,
        compiler_params=_sc_params,
        out_type=[sds((NEPAD,), f32), sds((NPAD,), f32), sds((NPAD,), f32)],
        mesh=_mesh,
        scratch_types=[
            pltpu.VMEM_SHARED((NPAD,), f32),
            pltpu.VMEM((NPAD,), f32), pltpu.VMEM((NPAD,), f32),
            pltpu.VMEM((CH,), _i32), pltpu.VMEM((CH,), _i32),
            pltpu.VMEM((CH,), f32), pltpu.VMEM((CH,), f32),
        ],
    )(srcp, dstp, p, s_p0, s_p1, zn)

    dis = pl.kernel(
        _dis_body,
        compiler_params=_sc_params,
        out_type=sds((NPAD,), f32),
        mesh=_mesh,
        scratch_types=[pltpu.VMEM((NWSL,), f32), pltpu.VMEM((NWSL,), f32)],
    )(deg_p0, deg_p1)

    loop_idx = jnp.arange(N, dtype=_i32)
    srcx = jnp.concatenate([src, loop_idx, jnp.zeros((NE2PAD - NE2,), _i32)])
    dstx = jnp.concatenate([dst, loop_idx, jnp.zeros((NE2PAD - NE2,), _i32)])
    wx = jnp.concatenate([w[:NE], jnp.ones((N,), f32), jnp.zeros((NE2PAD - NE2,), f32)])

    agg_p0, agg_p1, normx = pl.kernel(
        _spmv_body,
        compiler_params=_sc_params,
        out_type=[sds((NPAD, TP), f32), sds((NPAD, TP), f32), sds((NE2PAD,), f32)],
        mesh=_mesh,
        scratch_types=[
            pltpu.VMEM_SHARED((NPAD, TP), f32),
            pltpu.VMEM((NPAD,), f32),
            pltpu.VMEM((NSL // 8, TP), f32),
            pltpu.VMEM((CH,), _i32), pltpu.VMEM((CH,), _i32),
            pltpu.VMEM((CH,), f32), pltpu.VMEM((CH,), f32),
            pltpu.VMEM((CH, TP), f32), pltpu.VMEM((CH * TP,), f32),
        ],
    )(srcx, dstx, wx, dis, x16, znt)

    BN = 1024
    grid = NPAD // BN
    hw = pl.pallas_call(
        _gru_body,
        grid=(grid,),
        in_specs=[
            pl.BlockSpec((BN, TP), lambda i: (i, 0)),
            pl.BlockSpec((BN, TP), lambda i: (i, 0)),
            pl.BlockSpec((1, HID), lambda i: (0, 0)),
            pl.BlockSpec((1, HID), lambda i: (0, 0)),
            pl.BlockSpec((HID, 3 * HID), lambda i: (0, 0)),
            pl.BlockSpec((HID, 3 * HID), lambda i: (0, 0)),
            pl.BlockSpec((1, 3 * HID), lambda i: (0, 0)),
            pl.BlockSpec((1, 3 * HID), lambda i: (0, 0)),
            pl.BlockSpec((HID, 1), lambda i: (0, 0)),
        ],
        out_specs=pl.BlockSpec((BN, 1), lambda i: (i, 0)),
        out_shape=sds((NPAD, 1), f32),
    )(agg_p0, agg_p1, W1.astype(f32), b1.reshape(1, HID).astype(f32),
      W_ih.T.astype(f32), W_hh.T.astype(f32),
      b_ih.reshape(1, 3 * HID).astype(f32), b_hh.reshape(1, 3 * HID).astype(f32),
      W2.astype(f32))

    out_p0, out_p1 = pl.kernel(
        _final_body,
        compiler_params=_sc_params,
        out_type=[sds((NPAD,), f32), sds((NPAD,), f32)],
        mesh=_mesh,
        scratch_types=[
            pltpu.VMEM_SHARED((NPAD,), f32),
            pltpu.VMEM((NPAD,), f32),
            pltpu.VMEM((CH,), _i32), pltpu.VMEM((CH,), _i32),
            pltpu.VMEM((CH,), f32), pltpu.VMEM((CH,), f32),
        ],
    )(srcx, dstx, normx, hw.reshape(NPAD), zn)

    RW = 128
    rows = NPAD // RW
    BR = 8
    out_g = pl.pallas_call(
        _comb_body,
        grid=(rows // BR,),
        in_specs=[
            pl.BlockSpec((BR, RW), lambda i: (i, 0)),
            pl.BlockSpec((BR, RW), lambda i: (i, 0)),
            pl.BlockSpec((1, 1), lambda i: (0, 0)),
        ],
        out_specs=pl.BlockSpec((BR, RW), lambda i: (i, 0)),
        out_shape=sds((rows, RW), f32),
    )(out_p0.reshape(rows, RW), out_p1.reshape(rows, RW),
      b2.reshape(1, 1).astype(f32))

    return out_g.reshape(NPAD)[:N].reshape(1, N)
